# Initial kernel scaffold; baseline (speedup 1.0000x reference)
import jax
import jax.numpy as jnp
from jax.experimental import pallas as pl


def kernel(x, edge_index, lin0_W, lin0_b, conv_W, out_W, out_b):
    raise NotImplementedError("write your pallas kernel here")


# speedup vs baseline: 6.3888x; 6.3888x over previous
import math
import functools

import jax
import jax.numpy as jnp
from jax import lax
from jax.experimental import pallas as pl
from jax.experimental.pallas import tpu as pltpu
from jax.experimental.pallas import tpu_sc as plsc

N = 10000
NPAD = 10240
E = 320000
EPAD = 327680
HID = 64
IN_DIM = 128
LAYERS = 32
NUM_CLASSES = 7
ALPHA = 0.1
THETA = 0.5

NC = 2
NS = 16
ROWS_PER_TILE = NPAD // NS
EROWS = EPAD // 128
EROWS_PER_W = EROWS // (NC * NS)


def _sc_edge_body(h_ref, src_ref, dst_ref, out_ref, idx_s, idx_d, rows, agg):
    c = lax.axis_index("c")
    s = lax.axis_index("s")
    wid = s * NC + c

    def _zrow(i, _):
        for j in range(HID // 16):
            rows[i, pl.ds(16 * j, 16)] = jnp.zeros((16,), jnp.float32)
        return 0
    lax.fori_loop(0, 128, _zrow, 0)
    row0 = s * ROWS_PER_TILE
    for t in range(ROWS_PER_TILE // 128):
        pltpu.sync_copy(rows, agg.at[pl.ds(row0 + t * 128, 128)])
    plsc.subcore_barrier()

    def _edge_batch(j, _):
        r = wid * EROWS_PER_W + j
        pltpu.sync_copy(src_ref.at[r], idx_s)
        pltpu.sync_copy(dst_ref.at[r], idx_d)
        pltpu.sync_copy(h_ref.at[idx_s], rows)
        pltpu.sync_copy(rows, agg.at[idx_d], add=True)
        return 0
    lax.fori_loop(0, EROWS_PER_W, _edge_batch, 0)
    plsc.subcore_barrier()

    pltpu.sync_copy(agg.at[pl.ds(row0, ROWS_PER_TILE)],
                    out_ref.at[c, pl.ds(row0, ROWS_PER_TILE)])


_sc_edge = pl.kernel(
    _sc_edge_body,
    out_type=jax.ShapeDtypeStruct((NC, NPAD, HID), jnp.float32),
    mesh=plsc.VectorSubcoreMesh(core_axis_name="c", subcore_axis_name="s"),
    compiler_params=pltpu.CompilerParams(use_tc_tiling_on_sc=False),
    scratch_types=[
        pltpu.VMEM((128,), jnp.int32),
        pltpu.VMEM((128,), jnp.int32),
        pltpu.VMEM((128, HID), jnp.float32),
        pltpu.VMEM_SHARED((NPAD, HID), jnp.float32),
    ],
)


def _tc_prologue_body(x_ref, w_ref, b_ref, deg_ref, h0_ref, ht_ref, dinv_ref):
    h0 = jnp.maximum(
        jnp.dot(x_ref[...], w_ref[...], preferred_element_type=jnp.float32)
        + b_ref[...][None, :], 0.0)
    deg = deg_ref[0] + deg_ref[1] + 1.0
    mask = lax.broadcasted_iota(jnp.int32, (NPAD, HID), 0) < N
    dinv = jnp.where(mask, lax.rsqrt(jnp.maximum(deg, 1.0)), 0.0)
    h0_ref[...] = h0
    ht_ref[...] = dinv * h0
    dinv_ref[...] = dinv


_tc_prologue = pl.pallas_call(
    _tc_prologue_body,
    out_shape=[
        jax.ShapeDtypeStruct((NPAD, HID), jnp.float32),
        jax.ShapeDtypeStruct((NPAD, HID), jnp.float32),
        jax.ShapeDtypeStruct((NPAD, HID), jnp.float32),
    ],
)


def _tc_layer_body(a_ref, ht_ref, h0_ref, dinv_ref, w_ref, h_out, ht_out):
    p = a_ref[0] + a_ref[1] + ht_ref[...]
    s = (1.0 - ALPHA) * (dinv_ref[...] * p) + ALPHA * h0_ref[...]
    h = jnp.maximum(
        jnp.dot(s, w_ref[...], preferred_element_type=jnp.float32), 0.0)
    h_out[...] = h
    ht_out[...] = dinv_ref[...] * h


_tc_layer = pl.pallas_call(
    _tc_layer_body,
    out_shape=[
        jax.ShapeDtypeStruct((NPAD, HID), jnp.float32),
        jax.ShapeDtypeStruct((NPAD, HID), jnp.float32),
    ],
)


def _tc_final_body(h_ref, w_ref, b_ref, o_ref):
    o_ref[...] = (
        jnp.dot(h_ref[...], w_ref[...], preferred_element_type=jnp.float32)
        + b_ref[...][None, :])


_tc_final = pl.pallas_call(
    _tc_final_body,
    out_shape=jax.ShapeDtypeStruct((NPAD, 128), jnp.float32),
)


def kernel(x, edge_index, lin0_W, lin0_b, conv_W, out_W, out_b):
    ei = edge_index.astype(jnp.int32)
    pad = jnp.full((EPAD - E,), N, jnp.int32)
    src = jnp.concatenate([ei[0], pad]).reshape(EROWS, 128)
    dst = jnp.concatenate([ei[1], pad]).reshape(EROWS, 128)
    x_pad = jnp.pad(x, ((0, NPAD - N), (0, 0)))
    out_Wp = jnp.pad(out_W, ((0, 0), (0, 128 - NUM_CLASSES)))
    out_bp = jnp.pad(out_b, (0, 128 - NUM_CLASSES))

    ones_rep = jnp.ones((NPAD, HID), jnp.float32)
    deg = _sc_edge(ones_rep, src, dst)
    h0, ht, dinv = _tc_prologue(x_pad, lin0_W, lin0_b, deg)

    eye = jnp.eye(HID, dtype=jnp.float32)
    h = h0
    for i in range(LAYERS):
        beta = math.log(THETA / (i + 1) + 1.0)
        w_eff = (1.0 - beta) * eye + beta * conv_W[i]
        a = _sc_edge(ht, src, dst)
        h, ht = _tc_layer(a, ht, h0, dinv, w_eff)

    out = _tc_final(h, out_Wp, out_bp)
    return out[:N, :NUM_CLASSES]
